# Initial kernel scaffold; baseline (speedup 1.0000x reference)
import jax
import jax.numpy as jnp
from jax.experimental import pallas as pl


def kernel(x, edge_index, W1, W_out, b_out):
    raise NotImplementedError("write your pallas kernel here")


# speedup vs baseline: 1.7830x; 1.7830x over previous
import functools

import jax
import jax.numpy as jnp
from jax.experimental import pallas as pl
from jax.experimental.pallas import tpu as pltpu


def _sg_mm_body(part_ref, h_ref, norm_ref, w_ref, out_ref):
    a = part_ref[...] + h_ref[...]
    prod = jnp.dot(a, w_ref[...], preferred_element_type=jnp.float32)
    out_ref[...] = prod * norm_ref[...]


def _head_body(x2_ref, wt_ref, b_ref, out_ref, feat_ref):
    x2 = x2_ref[...]
    s = jnp.sum(x2 * x2, axis=1, keepdims=True)
    inv = jax.lax.rsqrt(jnp.maximum(s, 1e-24))
    feat = x2 * inv
    feat_ref[...] = feat
    out_ref[...] = (
        jnp.dot(feat, wt_ref[...], preferred_element_type=jnp.float32) + b_ref[...]
    )


def kernel(x, edge_index, W1, W_out, b_out):
    n, d = x.shape
    c = W_out.shape[0]
    src = edge_index[0]
    dst = edge_index[1]

    cnt = jnp.zeros((n,), jnp.int32).at[dst].add(1)
    norm = jax.lax.rsqrt((cnt + 1).astype(x.dtype))

    h = x * norm[:, None]
    part = jnp.zeros((n, d), x.dtype).at[dst].add(jnp.take(h, src, axis=0))

    bm = 1000
    grid = (n // bm,)
    h2 = pl.pallas_call(
        _sg_mm_body,
        grid=grid,
        in_specs=[
            pl.BlockSpec((bm, d), lambda i: (i, 0)),
            pl.BlockSpec((bm, d), lambda i: (i, 0)),
            pl.BlockSpec((bm, 1), lambda i: (i, 0)),
            pl.BlockSpec((d, d), lambda i: (0, 0)),
        ],
        out_specs=pl.BlockSpec((bm, d), lambda i: (i, 0)),
        out_shape=jax.ShapeDtypeStruct((n, d), x.dtype),
    )(part, h, norm[:, None], W1)

    present = (cnt > 0).astype(jnp.int32)
    ranks = jnp.cumsum(present) - 1
    u = (
        jnp.zeros((n,), dst.dtype)
        .at[jnp.where(present > 0, ranks, n)]
        .set(jnp.arange(n, dtype=dst.dtype), mode="drop")
    )

    x2 = jnp.take(h2, u, axis=0)

    out, feat = pl.pallas_call(
        _head_body,
        grid=grid,
        in_specs=[
            pl.BlockSpec((bm, d), lambda i: (i, 0)),
            pl.BlockSpec((d, c), lambda i: (0, 0)),
            pl.BlockSpec((1, c), lambda i: (0, 0)),
        ],
        out_specs=[
            pl.BlockSpec((bm, c), lambda i: (i, 0)),
            pl.BlockSpec((bm, d), lambda i: (i, 0)),
        ],
        out_shape=[
            jax.ShapeDtypeStruct((n, c), x.dtype),
            jax.ShapeDtypeStruct((n, d), x.dtype),
        ],
    )(x2, W_out.T, b_out[None, :])
    return (out, feat)


# speedup vs baseline: 11.7427x; 6.5860x over previous
import functools

import jax
import jax.numpy as jnp
from jax import lax
from jax.experimental import pallas as pl
from jax.experimental.pallas import tpu as pltpu
from jax.experimental.pallas import tpu_sc as plsc

_NC, _NS = 2, 16
_CHUNK = 125


def _sc_aggregate(h, src2d, dst2d, z):
    n, d = h.shape
    n_pad = ((n + 8 * _NS - 1) // (8 * _NS)) * (8 * _NS)
    rows_total = src2d.shape[0]
    rows_per_tile = rows_total // (_NC * _NS)
    stripe = n_pad // _NS
    assert z.shape[0] == stripe

    mesh = plsc.VectorSubcoreMesh(core_axis_name="c", subcore_axis_name="s")

    @functools.partial(
        pl.kernel,
        out_type=jax.ShapeDtypeStruct((_NC, n_pad, d), jnp.float32),
        mesh=mesh,
        scratch_types=[
            pltpu.VMEM((rows_per_tile, _CHUNK), jnp.int32),
            pltpu.VMEM((rows_per_tile, _CHUNK), jnp.int32),
            pltpu.VMEM((_CHUNK, d), jnp.float32),
            pltpu.VMEM_SHARED((n_pad, d), jnp.float32),
            pltpu.SemaphoreType.DMA,
        ],
    )
    def agg(h_hbm, src_hbm, dst_hbm, z_hbm, out_hbm,
            src_v, dst_v, rows_v, acc_sh, sem):
        cid = lax.axis_index("c")
        sid = lax.axis_index("s")
        base = sid * stripe
        pltpu.sync_copy(z_hbm, acc_sh.at[pl.ds(base, stripe)])
        row0 = (cid * _NS + sid) * rows_per_tile
        pltpu.sync_copy(src_hbm.at[pl.ds(row0, rows_per_tile)], src_v)
        pltpu.sync_copy(dst_hbm.at[pl.ds(row0, rows_per_tile)], dst_v)
        plsc.subcore_barrier()

        @pl.loop(0, rows_per_tile)
        def _(j):
            pltpu.async_copy(h_hbm.at[src_v.at[j]], rows_v, sem).wait()
            pltpu.sync_copy(rows_v, acc_sh.at[dst_v.at[j]], add=True)

        plsc.subcore_barrier()
        sl = pl.ds(base, stripe)
        pltpu.sync_copy(acc_sh.at[sl], out_hbm.at[cid].at[sl])

    return agg(h, src2d, dst2d, z)


def _sg_mm_body(parts_ref, h_ref, norm_ref, w_ref, out_ref):
    a = parts_ref[0] + parts_ref[1] + h_ref[...]
    prod = jnp.dot(a, w_ref[...], preferred_element_type=jnp.float32)
    out_ref[...] = prod * norm_ref[...]


def _head_body(x2_ref, wt_ref, b_ref, out_ref, feat_ref):
    x2 = x2_ref[...]
    s = jnp.sum(x2 * x2, axis=1, keepdims=True)
    inv = jax.lax.rsqrt(jnp.maximum(s, 1e-24))
    feat = x2 * inv
    feat_ref[...] = feat
    out_ref[...] = (
        jnp.dot(feat, wt_ref[...], preferred_element_type=jnp.float32) + b_ref[...]
    )


def kernel(x, edge_index, W1, W_out, b_out):
    n, d = x.shape
    c = W_out.shape[0]
    src = edge_index[0]
    dst = edge_index[1]

    cnt = jnp.zeros((n,), jnp.int32).at[dst].add(1)
    norm = jax.lax.rsqrt((cnt + 1).astype(x.dtype))

    h = x * norm[:, None]
    e = src.shape[0]
    n_pad = ((n + 8 * _NS - 1) // (8 * _NS)) * (8 * _NS)
    parts = _sc_aggregate(
        h,
        src.reshape(e // _CHUNK, _CHUNK),
        dst.reshape(e // _CHUNK, _CHUNK),
        jnp.zeros((n_pad // _NS, d), x.dtype),
    )

    bm = 1000
    grid = (n // bm,)
    h2 = pl.pallas_call(
        _sg_mm_body,
        grid=grid,
        in_specs=[
            pl.BlockSpec((2, bm, d), lambda i: (0, i, 0)),
            pl.BlockSpec((bm, d), lambda i: (i, 0)),
            pl.BlockSpec((bm, 1), lambda i: (i, 0)),
            pl.BlockSpec((d, d), lambda i: (0, 0)),
        ],
        out_specs=pl.BlockSpec((bm, d), lambda i: (i, 0)),
        out_shape=jax.ShapeDtypeStruct((n, d), x.dtype),
    )(parts, h, norm[:, None], W1)

    present = (cnt > 0).astype(jnp.int32)
    ranks = jnp.cumsum(present) - 1
    u = (
        jnp.zeros((n,), dst.dtype)
        .at[jnp.where(present > 0, ranks, n)]
        .set(jnp.arange(n, dtype=dst.dtype), mode="drop")
    )

    x2 = jnp.take(h2, u, axis=0)

    out, feat = pl.pallas_call(
        _head_body,
        grid=grid,
        in_specs=[
            pl.BlockSpec((bm, d), lambda i: (i, 0)),
            pl.BlockSpec((d, c), lambda i: (0, 0)),
            pl.BlockSpec((1, c), lambda i: (0, 0)),
        ],
        out_specs=[
            pl.BlockSpec((bm, c), lambda i: (i, 0)),
            pl.BlockSpec((bm, d), lambda i: (i, 0)),
        ],
        out_shape=[
            jax.ShapeDtypeStruct((n, c), x.dtype),
            jax.ShapeDtypeStruct((n, d), x.dtype),
        ],
    )(x2, W_out.T, b_out[None, :])
    return (out, feat)
